# baseline (speedup 1.0000x reference)
import jax
import jax.numpy as jnp
from jax.experimental import pallas as pl
from jax.experimental.pallas import tpu as pltpu

E_B = 0.05
E_N = 0.006
D_DECAY = 0.995
INPUT_DIM = 150528
BATCH = 64
M = BATCH + 2

_NC = 8
_KC = INPUT_DIM // _NC


_DN_T = (((1,), (1,)), ((), ()))
_DN = (((1,), (0,)), ((), ()))


def _dot(a, b, dn):
    return jax.lax.dot_general(a, b, dn, preferred_element_type=jnp.float32)


def _fused_kernel(n_ref, x_ref, out_ref, err_ref,
                  hi_ref, g_ref, c_ref):
    ph = pl.program_id(0)
    j = pl.program_id(1)

    @pl.when(ph == 0)
    def _():
        x = x_ref[...]
        n = n_ref[...]
        hi_x = x.astype(jnp.bfloat16)
        hi_n = n.astype(jnp.bfloat16)
        hi_ref[j] = hi_x
        hix32 = hi_x.astype(jnp.float32)
        hin32 = hi_n.astype(jnp.float32)
        hi = jnp.concatenate([hi_x, hi_n], axis=0)
        lo = jnp.concatenate(
            [(x - hix32).astype(jnp.bfloat16),
             (n - hin32).astype(jnp.bfloat16)], axis=0)
        d1 = _dot(hi, hi, _DN_T)
        d2 = _dot(hi, lo, _DN_T)
        g = d1 + d2 + d2.T

        @pl.when(j == 0)
        def _():
            g_ref[...] = g

        @pl.when(j != 0)
        def _():
            g_ref[...] += g

    @pl.when((ph == 1) & (j == 0))
    def _():
        f32 = jnp.float32
        one = jnp.float32(1.0)
        lane = jax.lax.broadcasted_iota(jnp.int32, (1, M), 1)
        dE = E_B - E_N
        gd = [g_ref[p, p] for p in range(BATCH)]
        gb1 = [None] + [g_ref[p - 1, p] for p in range(1, BATCH)]
        gb2 = [None, None] + [g_ref[p - 2, p] for p in range(2, BATCH)]
        c0 = (lane == BATCH).astype(f32)
        c1 = (lane == BATCH + 1).astype(f32)
        cg0 = g_ref[BATCH:BATCH + 1, :]
        cg1 = g_ref[BATCH + 1:BATCH + 2, :]
        n0sq = g_ref[BATCH, BATCH]
        n1sq = g_ref[BATCH + 1, BATCH + 1]
        n01 = g_ref[BATCH, BATCH + 1]
        zero = jnp.float32(0.0)
        e0 = e1 = zero
        q0 = g_ref[BATCH, 0]
        q1 = g_ref[BATCH + 1, 0]
        Z0 = g_ref[BATCH, 1]
        Z1 = g_ref[BATCH + 1, 1]
        w_prev = r0_prev = r1_prev = None

        for p in range(BATCH):
            if p + 2 < BATCH:
                newZ0 = cg0[0, p + 2]
                newZ1 = cg1[0, p + 2]
            d0 = n0sq - 2.0 * q0 + gd[p]
            d1 = n1sq - 2.0 * q1 + gd[p]
            w = jnp.where(d0 <= d1, one, zero)
            r0 = E_N + w * dE
            r1 = E_B - w * dE
            qb = q1 + w * (q0 - q1)
            qs = q0 + w * (q1 - q0)
            nb = n1sq + w * (n0sq - n1sq)
            ns = n0sq + w * (n1sq - n0sq)
            db = d1 + w * (d0 - d1)
            nb_new = ((1.0 - E_B) ** 2 * nb + 2.0 * E_B * (1.0 - E_B) * qb
                      + E_B * E_B * gd[p])
            ns_new = ns + 2.0 * E_N * (qs - n01) + E_N * E_N * db
            nbx = (1.0 - E_B) * qb + E_B * gd[p]
            nbb = (1.0 - E_B) * nb + E_B * qb
            n01 = (1.0 - E_B) * n01 + E_B * qs + E_N * (nbx - nbb)
            n0sq = ns_new + w * (nb_new - ns_new)
            n1sq = nb_new + w * (ns_new - nb_new)
            e0 = (e0 + w * db) * D_DECAY
            e1 = (e1 + db - w * db) * D_DECAY
            if p + 1 < BATCH:
                if p >= 1:
                    cgbZ = Z1 + w_prev * (Z0 - Z1)
                    Z0 = Z0 + r0_prev * (gb2[p + 1] - cgbZ)
                    Z1 = Z1 + r1_prev * (gb2[p + 1] - cgbZ)
                cgbZ2 = Z1 + w * (Z0 - Z1)
                q0n = Z0 + r0 * (gb1[p + 1] - cgbZ2)
                q1n = Z1 + r1 * (gb1[p + 1] - cgbZ2)
            onehot = (lane == p).astype(f32)
            gp = g_ref[p:p + 1, :]
            cb = c1 + w * (c0 - c1)
            cgb = cg1 + w * (cg0 - cg1)
            c0 = c0 + r0 * (onehot - cb)
            c1 = c1 + r1 * (onehot - cb)
            cg0 = cg0 + r0 * (gp - cgb)
            cg1 = cg1 + r1 * (gp - cgb)
            w_prev, r0_prev, r1_prev = w, r0, r1
            if p + 1 < BATCH:
                q0, q1 = q0n, q1n
            if p + 2 < BATCH:
                Z0, Z1 = newZ0, newZ1
        err_ref[...] = ((lane == 0).astype(f32) * e0
                        + (lane == 1).astype(f32) * e1)
        c_ref[...] = jnp.concatenate(
            [c0, c1, jnp.zeros((6, M), jnp.float32)], axis=0)

    @pl.when(ph == 1)
    def _():
        cm = c_ref[...]
        hi_c = cm.astype(jnp.bfloat16)
        lo_c = (cm - hi_c.astype(jnp.float32)).astype(jnp.bfloat16)
        hi_x = hi_ref[j]
        n = n_ref[...]
        hi_n = n.astype(jnp.bfloat16)
        lo_n = (n - hi_n.astype(jnp.float32)).astype(jnp.bfloat16)
        a_img = jnp.concatenate(
            [hi_c[:, 0:BATCH], lo_c[:, 0:BATCH]], axis=0)
        t16 = _dot(a_img, hi_x, _DN)
        a_n = jnp.concatenate(
            [hi_c[:, BATCH:M], lo_c[:, BATCH:M], hi_c[:, BATCH:M]],
            axis=1)
        b_n = jnp.concatenate([hi_n, hi_n, lo_n], axis=0)
        out8 = t16[0:8, :] + t16[8:16, :] + _dot(a_n, b_n, _DN)
        out_ref[...] = out8[0:2, :]


def kernel(images, labels, nodes, local_error, edges):
    del labels
    nodes_out, err_row = pl.pallas_call(
        _fused_kernel,
        grid=(2, _NC),
        in_specs=[
            pl.BlockSpec((2, _KC), lambda p, j: (0, j)),
            pl.BlockSpec((BATCH, _KC),
                         lambda p, j: (0, j * (1 - p) + (_NC - 1) * p)),
        ],
        out_specs=[
            pl.BlockSpec((2, _KC), lambda p, j: (0, j * p)),
            pl.BlockSpec((1, M), lambda p, j: (0, 0)),
        ],
        out_shape=[
            jax.ShapeDtypeStruct((2, INPUT_DIM), jnp.float32),
            jax.ShapeDtypeStruct((1, M), jnp.float32),
        ],
        scratch_shapes=[
            pltpu.VMEM((_NC, BATCH, _KC), jnp.bfloat16),
            pltpu.VMEM((M, M), jnp.float32),
            pltpu.VMEM((8, M), jnp.float32),
        ],
    )(nodes, images)

    local_error_out = err_row[0, 0:2] + local_error * (D_DECAY ** BATCH)
    return nodes_out, local_error_out, edges


# speedup vs baseline: 1.0450x; 1.0450x over previous
import jax
import jax.numpy as jnp
from jax.experimental import pallas as pl
from jax.experimental.pallas import tpu as pltpu

E_B = 0.05
E_N = 0.006
D_DECAY = 0.995
INPUT_DIM = 150528
BATCH = 64
M = BATCH + 2

_NC = 3
_KC = INPUT_DIM // _NC


_DN_T = (((1,), (1,)), ((), ()))
_DN = (((1,), (0,)), ((), ()))


def _dot(a, b, dn):
    return jax.lax.dot_general(a, b, dn, preferred_element_type=jnp.float32)


def _fused_kernel(n_ref, x_ref, out_ref, err_ref,
                  hi_ref, g_ref, c_ref):
    ph = pl.program_id(0)
    j = pl.program_id(1)

    @pl.when(ph == 0)
    def _():
        x = x_ref[...]
        n = n_ref[...]
        hi_x = x.astype(jnp.bfloat16)
        hi_n = n.astype(jnp.bfloat16)
        hi_ref[j] = hi_x
        hix32 = hi_x.astype(jnp.float32)
        hin32 = hi_n.astype(jnp.float32)
        hi = jnp.concatenate([hi_x, hi_n], axis=0)
        lo = jnp.concatenate(
            [(x - hix32).astype(jnp.bfloat16),
             (n - hin32).astype(jnp.bfloat16)], axis=0)
        d1 = _dot(hi, hi, _DN_T)
        d2 = _dot(hi, lo, _DN_T)
        g = d1 + d2 + d2.T

        @pl.when(j == 0)
        def _():
            g_ref[...] = g

        @pl.when(j != 0)
        def _():
            g_ref[...] += g

    @pl.when((ph == 1) & (j == 0))
    def _():
        f32 = jnp.float32
        one = jnp.float32(1.0)
        lane = jax.lax.broadcasted_iota(jnp.int32, (1, M), 1)
        dE = E_B - E_N
        gd = [g_ref[p, p] for p in range(BATCH)]
        gb1 = [None] + [g_ref[p - 1, p] for p in range(1, BATCH)]
        gb2 = [None, None] + [g_ref[p - 2, p] for p in range(2, BATCH)]
        c0 = (lane == BATCH).astype(f32)
        c1 = (lane == BATCH + 1).astype(f32)
        cg0 = g_ref[BATCH:BATCH + 1, :]
        cg1 = g_ref[BATCH + 1:BATCH + 2, :]
        n0sq = g_ref[BATCH, BATCH]
        n1sq = g_ref[BATCH + 1, BATCH + 1]
        n01 = g_ref[BATCH, BATCH + 1]
        zero = jnp.float32(0.0)
        e0 = e1 = zero
        q0 = g_ref[BATCH, 0]
        q1 = g_ref[BATCH + 1, 0]
        Z0 = g_ref[BATCH, 1]
        Z1 = g_ref[BATCH + 1, 1]
        w_prev = r0_prev = r1_prev = None

        for p in range(BATCH):
            if p + 2 < BATCH:
                newZ0 = cg0[0, p + 2]
                newZ1 = cg1[0, p + 2]
            d0 = n0sq - 2.0 * q0 + gd[p]
            d1 = n1sq - 2.0 * q1 + gd[p]
            w = jnp.where(d0 <= d1, one, zero)
            r0 = E_N + w * dE
            r1 = E_B - w * dE
            qb = q1 + w * (q0 - q1)
            qs = q0 + w * (q1 - q0)
            nb = n1sq + w * (n0sq - n1sq)
            ns = n0sq + w * (n1sq - n0sq)
            db = d1 + w * (d0 - d1)
            nb_new = ((1.0 - E_B) ** 2 * nb + 2.0 * E_B * (1.0 - E_B) * qb
                      + E_B * E_B * gd[p])
            ns_new = ns + 2.0 * E_N * (qs - n01) + E_N * E_N * db
            nbx = (1.0 - E_B) * qb + E_B * gd[p]
            nbb = (1.0 - E_B) * nb + E_B * qb
            n01 = (1.0 - E_B) * n01 + E_B * qs + E_N * (nbx - nbb)
            n0sq = ns_new + w * (nb_new - ns_new)
            n1sq = nb_new + w * (ns_new - nb_new)
            e0 = (e0 + w * db) * D_DECAY
            e1 = (e1 + db - w * db) * D_DECAY
            if p + 1 < BATCH:
                if p >= 1:
                    cgbZ = Z1 + w_prev * (Z0 - Z1)
                    Z0 = Z0 + r0_prev * (gb2[p + 1] - cgbZ)
                    Z1 = Z1 + r1_prev * (gb2[p + 1] - cgbZ)
                cgbZ2 = Z1 + w * (Z0 - Z1)
                q0n = Z0 + r0 * (gb1[p + 1] - cgbZ2)
                q1n = Z1 + r1 * (gb1[p + 1] - cgbZ2)
            onehot = (lane == p).astype(f32)
            gp = g_ref[p:p + 1, :]
            cb = c1 + w * (c0 - c1)
            cgb = cg1 + w * (cg0 - cg1)
            c0 = c0 + r0 * (onehot - cb)
            c1 = c1 + r1 * (onehot - cb)
            cg0 = cg0 + r0 * (gp - cgb)
            cg1 = cg1 + r1 * (gp - cgb)
            w_prev, r0_prev, r1_prev = w, r0, r1
            if p + 1 < BATCH:
                q0, q1 = q0n, q1n
            if p + 2 < BATCH:
                Z0, Z1 = newZ0, newZ1
        err_ref[...] = ((lane == 0).astype(f32) * e0
                        + (lane == 1).astype(f32) * e1)
        c_ref[...] = jnp.concatenate(
            [c0, c1, jnp.zeros((6, M), jnp.float32)], axis=0)

    @pl.when(ph == 1)
    def _():
        cm = c_ref[...]
        hi_c = cm.astype(jnp.bfloat16)
        lo_c = (cm - hi_c.astype(jnp.float32)).astype(jnp.bfloat16)
        hi_x = hi_ref[j]
        n = n_ref[...]
        hi_n = n.astype(jnp.bfloat16)
        lo_n = (n - hi_n.astype(jnp.float32)).astype(jnp.bfloat16)
        a_img = jnp.concatenate(
            [hi_c[:, 0:BATCH], lo_c[:, 0:BATCH]], axis=0)
        t16 = _dot(a_img, hi_x, _DN)
        a_n = jnp.concatenate(
            [hi_c[:, BATCH:M], lo_c[:, BATCH:M], hi_c[:, BATCH:M]],
            axis=1)
        b_n = jnp.concatenate([hi_n, hi_n, lo_n], axis=0)
        out8 = t16[0:8, :] + t16[8:16, :] + _dot(a_n, b_n, _DN)
        out_ref[...] = out8[0:2, :]


def kernel(images, labels, nodes, local_error, edges):
    del labels
    nodes_out, err_row = pl.pallas_call(
        _fused_kernel,
        grid=(2, _NC),
        in_specs=[
            pl.BlockSpec((2, _KC), lambda p, j: (0, j)),
            pl.BlockSpec((BATCH, _KC),
                         lambda p, j: (0, j * (1 - p) + (_NC - 1) * p)),
        ],
        out_specs=[
            pl.BlockSpec((2, _KC), lambda p, j: (0, j * p)),
            pl.BlockSpec((1, M), lambda p, j: (0, 0)),
        ],
        out_shape=[
            jax.ShapeDtypeStruct((2, INPUT_DIM), jnp.float32),
            jax.ShapeDtypeStruct((1, M), jnp.float32),
        ],
        scratch_shapes=[
            pltpu.VMEM((_NC, BATCH, _KC), jnp.bfloat16),
            pltpu.VMEM((M, M), jnp.float32),
            pltpu.VMEM((8, M), jnp.float32),
        ],
    )(nodes, images)

    local_error_out = err_row[0, 0:2] + local_error * (D_DECAY ** BATCH)
    return nodes_out, local_error_out, edges


# speedup vs baseline: 1.0595x; 1.0138x over previous
import jax
import jax.numpy as jnp
from jax.experimental import pallas as pl
from jax.experimental.pallas import tpu as pltpu

E_B = 0.05
E_N = 0.006
D_DECAY = 0.995
INPUT_DIM = 150528
BATCH = 64
M = BATCH + 2

_NC = 4
_KC = INPUT_DIM // _NC


_DN_T = (((1,), (1,)), ((), ()))
_DN = (((1,), (0,)), ((), ()))


def _dot(a, b, dn):
    return jax.lax.dot_general(a, b, dn, preferred_element_type=jnp.float32)


def _fused_kernel(n_ref, x_ref, out_ref, err_ref,
                  hi_ref, g_ref, c_ref):
    ph = pl.program_id(0)
    j = pl.program_id(1)

    @pl.when(ph == 0)
    def _():
        x = x_ref[...]
        n = n_ref[...]
        hi_x = x.astype(jnp.bfloat16)
        hi_n = n.astype(jnp.bfloat16)
        hi_ref[j] = hi_x
        hix32 = hi_x.astype(jnp.float32)
        hin32 = hi_n.astype(jnp.float32)
        hi = jnp.concatenate([hi_x, hi_n], axis=0)
        lo = jnp.concatenate(
            [(x - hix32).astype(jnp.bfloat16),
             (n - hin32).astype(jnp.bfloat16)], axis=0)
        d1 = _dot(hi, hi, _DN_T)
        d2 = _dot(hi, lo, _DN_T)
        g = d1 + d2 + d2.T

        @pl.when(j == 0)
        def _():
            g_ref[...] = g

        @pl.when(j != 0)
        def _():
            g_ref[...] += g

    @pl.when((ph == 1) & (j == 0))
    def _():
        f32 = jnp.float32
        one = jnp.float32(1.0)
        lane = jax.lax.broadcasted_iota(jnp.int32, (1, M), 1)
        dE = E_B - E_N
        gd = [g_ref[p, p] for p in range(BATCH)]
        gb1 = [None] + [g_ref[p - 1, p] for p in range(1, BATCH)]
        gb2 = [None, None] + [g_ref[p - 2, p] for p in range(2, BATCH)]
        c0 = (lane == BATCH).astype(f32)
        c1 = (lane == BATCH + 1).astype(f32)
        cg0 = g_ref[BATCH:BATCH + 1, :]
        cg1 = g_ref[BATCH + 1:BATCH + 2, :]
        n0sq = g_ref[BATCH, BATCH]
        n1sq = g_ref[BATCH + 1, BATCH + 1]
        n01 = g_ref[BATCH, BATCH + 1]
        zero = jnp.float32(0.0)
        e0 = e1 = zero
        q0 = g_ref[BATCH, 0]
        q1 = g_ref[BATCH + 1, 0]
        Z0 = g_ref[BATCH, 1]
        Z1 = g_ref[BATCH + 1, 1]
        w_prev = r0_prev = r1_prev = None

        for p in range(BATCH):
            if p + 2 < BATCH:
                newZ0 = cg0[0, p + 2]
                newZ1 = cg1[0, p + 2]
            d0 = n0sq - 2.0 * q0 + gd[p]
            d1 = n1sq - 2.0 * q1 + gd[p]
            w = jnp.where(d0 <= d1, one, zero)
            r0 = E_N + w * dE
            r1 = E_B - w * dE
            qb = q1 + w * (q0 - q1)
            qs = q0 + w * (q1 - q0)
            nb = n1sq + w * (n0sq - n1sq)
            ns = n0sq + w * (n1sq - n0sq)
            db = d1 + w * (d0 - d1)
            nb_new = ((1.0 - E_B) ** 2 * nb + 2.0 * E_B * (1.0 - E_B) * qb
                      + E_B * E_B * gd[p])
            ns_new = ns + 2.0 * E_N * (qs - n01) + E_N * E_N * db
            nbx = (1.0 - E_B) * qb + E_B * gd[p]
            nbb = (1.0 - E_B) * nb + E_B * qb
            n01 = (1.0 - E_B) * n01 + E_B * qs + E_N * (nbx - nbb)
            n0sq = ns_new + w * (nb_new - ns_new)
            n1sq = nb_new + w * (ns_new - nb_new)
            e0 = (e0 + w * db) * D_DECAY
            e1 = (e1 + db - w * db) * D_DECAY
            if p + 1 < BATCH:
                if p >= 1:
                    cgbZ = Z1 + w_prev * (Z0 - Z1)
                    Z0 = Z0 + r0_prev * (gb2[p + 1] - cgbZ)
                    Z1 = Z1 + r1_prev * (gb2[p + 1] - cgbZ)
                cgbZ2 = Z1 + w * (Z0 - Z1)
                q0n = Z0 + r0 * (gb1[p + 1] - cgbZ2)
                q1n = Z1 + r1 * (gb1[p + 1] - cgbZ2)
            onehot = (lane == p).astype(f32)
            gp = g_ref[p:p + 1, :]
            cb = c1 + w * (c0 - c1)
            cgb = cg1 + w * (cg0 - cg1)
            c0 = c0 + r0 * (onehot - cb)
            c1 = c1 + r1 * (onehot - cb)
            cg0 = cg0 + r0 * (gp - cgb)
            cg1 = cg1 + r1 * (gp - cgb)
            w_prev, r0_prev, r1_prev = w, r0, r1
            if p + 1 < BATCH:
                q0, q1 = q0n, q1n
            if p + 2 < BATCH:
                Z0, Z1 = newZ0, newZ1
        err_ref[...] = ((lane == 0).astype(f32) * e0
                        + (lane == 1).astype(f32) * e1)
        c_ref[...] = jnp.concatenate(
            [c0, c1, jnp.zeros((6, M), jnp.float32)], axis=0)

    @pl.when(ph == 1)
    def _():
        cm = c_ref[...]
        hi_c = cm.astype(jnp.bfloat16)
        lo_c = (cm - hi_c.astype(jnp.float32)).astype(jnp.bfloat16)
        hi_x = hi_ref[j]
        n = n_ref[...]
        hi_n = n.astype(jnp.bfloat16)
        lo_n = (n - hi_n.astype(jnp.float32)).astype(jnp.bfloat16)
        a_n = jnp.concatenate(
            [hi_c[:, BATCH:M], lo_c[:, BATCH:M], hi_c[:, BATCH:M]],
            axis=1)
        b_n = jnp.concatenate([hi_n, hi_n, lo_n], axis=0)
        out8 = _dot(hi_c[:, 0:BATCH], hi_x, _DN) + _dot(a_n, b_n, _DN)
        out_ref[...] = out8[0:2, :]


def kernel(images, labels, nodes, local_error, edges):
    del labels
    nodes_out, err_row = pl.pallas_call(
        _fused_kernel,
        grid=(2, _NC),
        in_specs=[
            pl.BlockSpec((2, _KC), lambda p, j: (0, j)),
            pl.BlockSpec((BATCH, _KC),
                         lambda p, j: (0, j * (1 - p) + (_NC - 1) * p)),
        ],
        out_specs=[
            pl.BlockSpec((2, _KC), lambda p, j: (0, j * p)),
            pl.BlockSpec((1, M), lambda p, j: (0, 0)),
        ],
        out_shape=[
            jax.ShapeDtypeStruct((2, INPUT_DIM), jnp.float32),
            jax.ShapeDtypeStruct((1, M), jnp.float32),
        ],
        scratch_shapes=[
            pltpu.VMEM((_NC, BATCH, _KC), jnp.bfloat16),
            pltpu.VMEM((M, M), jnp.float32),
            pltpu.VMEM((8, M), jnp.float32),
        ],
    )(nodes, images)

    local_error_out = err_row[0, 0:2] + local_error * (D_DECAY ** BATCH)
    return nodes_out, local_error_out, edges


# speedup vs baseline: 1.0625x; 1.0029x over previous
import jax
import jax.numpy as jnp
from jax.experimental import pallas as pl
from jax.experimental.pallas import tpu as pltpu

E_B = 0.05
E_N = 0.006
D_DECAY = 0.995
INPUT_DIM = 150528
BATCH = 64
M = BATCH + 2

_NC = 4
_KC = INPUT_DIM // _NC


_DN_T = (((1,), (1,)), ((), ()))
_DN = (((1,), (0,)), ((), ()))


def _dot(a, b, dn):
    return jax.lax.dot_general(a, b, dn, preferred_element_type=jnp.float32)


def _fused_kernel(n_ref, x_ref, out_ref, err_ref,
                  hi_ref, g_ref, c_ref):
    ph = pl.program_id(0)
    j = pl.program_id(1)

    @pl.when(ph == 0)
    def _():
        x = x_ref[...]
        n = n_ref[...]
        hi_x = x.astype(jnp.bfloat16)
        hi_n = n.astype(jnp.bfloat16)
        hi_ref[j] = hi_x
        hix32 = hi_x.astype(jnp.float32)
        hin32 = hi_n.astype(jnp.float32)
        hi = jnp.concatenate([hi_x, hi_n], axis=0)
        lo = jnp.concatenate(
            [(x - hix32).astype(jnp.bfloat16),
             (n - hin32).astype(jnp.bfloat16)], axis=0)
        d1 = _dot(hi, hi, _DN_T)
        d2 = _dot(hi, lo, _DN_T)
        g = d1 + d2 + d2.T

        @pl.when(j == 0)
        def _():
            g_ref[...] = g

        @pl.when(j != 0)
        def _():
            g_ref[...] += g

    @pl.when((ph == 1) & (j == 0))
    def _():
        f32 = jnp.float32
        one = jnp.float32(1.0)
        lane = jax.lax.broadcasted_iota(jnp.int32, (1, M), 1)
        dE = E_B - E_N
        gd = [g_ref[p, p] for p in range(BATCH)]
        gb1 = [None] + [g_ref[p - 1, p] for p in range(1, BATCH)]
        gb2 = [None, None] + [g_ref[p - 2, p] for p in range(2, BATCH)]
        c0 = (lane == BATCH).astype(f32)
        c1 = (lane == BATCH + 1).astype(f32)
        cg0 = g_ref[BATCH:BATCH + 1, :]
        cg1 = g_ref[BATCH + 1:BATCH + 2, :]
        n0sq = g_ref[BATCH, BATCH]
        n1sq = g_ref[BATCH + 1, BATCH + 1]
        n01 = g_ref[BATCH, BATCH + 1]
        zero = jnp.float32(0.0)
        e0 = e1 = zero
        q0 = g_ref[BATCH, 0]
        q1 = g_ref[BATCH + 1, 0]
        Z0 = g_ref[BATCH, 1]
        Z1 = g_ref[BATCH + 1, 1]
        w_prev = r0_prev = r1_prev = None

        for p in range(BATCH):
            if p + 2 < BATCH:
                newZ0 = cg0[0, p + 2]
                newZ1 = cg1[0, p + 2]
            d0 = n0sq - 2.0 * q0 + gd[p]
            d1 = n1sq - 2.0 * q1 + gd[p]
            w = jnp.where(d0 <= d1, one, zero)
            r0 = E_N + w * dE
            r1 = E_B - w * dE
            qb = q1 + w * (q0 - q1)
            qs = q0 + w * (q1 - q0)
            nb = n1sq + w * (n0sq - n1sq)
            ns = n0sq + w * (n1sq - n0sq)
            db = d1 + w * (d0 - d1)
            nb_new = ((1.0 - E_B) ** 2 * nb + 2.0 * E_B * (1.0 - E_B) * qb
                      + E_B * E_B * gd[p])
            ns_new = ns + 2.0 * E_N * (qs - n01) + E_N * E_N * db
            nbx = (1.0 - E_B) * qb + E_B * gd[p]
            nbb = (1.0 - E_B) * nb + E_B * qb
            n01 = (1.0 - E_B) * n01 + E_B * qs + E_N * (nbx - nbb)
            n0sq = ns_new + w * (nb_new - ns_new)
            n1sq = nb_new + w * (ns_new - nb_new)
            e0 = (e0 + w * db) * D_DECAY
            e1 = (e1 + db - w * db) * D_DECAY
            if p + 1 < BATCH:
                if p >= 1:
                    cgbZ = Z1 + w_prev * (Z0 - Z1)
                    Z0 = Z0 + r0_prev * (gb2[p + 1] - cgbZ)
                    Z1 = Z1 + r1_prev * (gb2[p + 1] - cgbZ)
                cgbZ2 = Z1 + w * (Z0 - Z1)
                q0n = Z0 + r0 * (gb1[p + 1] - cgbZ2)
                q1n = Z1 + r1 * (gb1[p + 1] - cgbZ2)
            onehot = (lane == p).astype(f32)
            gp = g_ref[p:p + 1, :]
            cb = c1 + w * (c0 - c1)
            cgb = cg1 + w * (cg0 - cg1)
            c0 = c0 + r0 * (onehot - cb)
            c1 = c1 + r1 * (onehot - cb)
            cg0 = cg0 + r0 * (gp - cgb)
            cg1 = cg1 + r1 * (gp - cgb)
            w_prev, r0_prev, r1_prev = w, r0, r1
            if p + 1 < BATCH:
                q0, q1 = q0n, q1n
            if p + 2 < BATCH:
                Z0, Z1 = newZ0, newZ1
        err_ref[...] = ((lane == 0).astype(f32) * e0
                        + (lane == 1).astype(f32) * e1)
        c_ref[...] = jnp.concatenate(
            [c0, c1, jnp.zeros((6, M), jnp.float32)], axis=0)

    @pl.when(ph == 1)
    def _():
        cm = c_ref[...]
        hi_c = cm.astype(jnp.bfloat16)
        lo_c = (cm - hi_c.astype(jnp.float32)).astype(jnp.bfloat16)
        hi_x = hi_ref[j]
        n = n_ref[...]
        hi_n = n.astype(jnp.bfloat16)
        lo_n = (n - hi_n.astype(jnp.float32)).astype(jnp.bfloat16)
        a_img = jnp.concatenate(
            [hi_c[:, 0:BATCH], lo_c[:, 0:BATCH]], axis=0)
        t16 = _dot(a_img, hi_x, _DN)
        a_n = jnp.concatenate(
            [hi_c[:, BATCH:M], lo_c[:, BATCH:M], hi_c[:, BATCH:M]],
            axis=1)
        b_n = jnp.concatenate([hi_n, hi_n, lo_n], axis=0)
        out8 = t16[0:8, :] + t16[8:16, :] + _dot(a_n, b_n, _DN)
        out_ref[...] = out8[0:2, :]


def kernel(images, labels, nodes, local_error, edges):
    del labels
    nodes_out, err_row = pl.pallas_call(
        _fused_kernel,
        grid=(2, _NC),
        in_specs=[
            pl.BlockSpec((2, _KC), lambda p, j: (0, j)),
            pl.BlockSpec((BATCH, _KC),
                         lambda p, j: (0, j * (1 - p) + (_NC - 1) * p)),
        ],
        out_specs=[
            pl.BlockSpec((2, _KC), lambda p, j: (0, j * p)),
            pl.BlockSpec((1, M), lambda p, j: (0, 0)),
        ],
        out_shape=[
            jax.ShapeDtypeStruct((2, INPUT_DIM), jnp.float32),
            jax.ShapeDtypeStruct((1, M), jnp.float32),
        ],
        scratch_shapes=[
            pltpu.VMEM((_NC, BATCH, _KC), jnp.bfloat16),
            pltpu.VMEM((M, M), jnp.float32),
            pltpu.VMEM((8, M), jnp.float32),
        ],
    )(nodes, images)

    local_error_out = err_row[0, 0:2] + local_error * (D_DECAY ** BATCH)
    return nodes_out, local_error_out, edges
